# baseline (speedup 1.0000x reference)
import functools

import jax
import jax.numpy as jnp
from jax import lax
from jax.experimental import pallas as pl
from jax.experimental.pallas import tpu as pltpu
from jax.experimental.pallas import tpu_sc as plsc

_N = 320000
_D = 128
_S = 10000

_B3 = 1280
_NB3 = _N // _B3
_SEGW = 64
_SEGP = 10240

_NC = 2
_NS = 16
_NW = _NC * _NS
_RPW = _N // _NW
_L = 16


_TW = _NB3 + _SEGP // _SEGW - 1 + 1


def _fused_body(blk_ref, base_ref, first_ref, feats_ref, w_ref, b_ref,
                ids_ref, scores_ref, acc_ref, m_ref, g_ref, erow_ref):
    t = pl.program_id(0)

    @pl.when(t == 0)
    def _():
        acc_ref[...] = jnp.zeros_like(acc_ref)
        g_ref[0] = -jnp.inf

    f = feats_ref[...]

    @pl.when(first_ref[t] == 1)
    def _():
        s = jnp.dot(f, w_ref[...],
                    preferred_element_type=jnp.float32) + b_ref[0, 0]
        scores_ref[...] = s
        bmax = jnp.max(s)
        g_old = g_ref[0]

        @pl.when(bmax > g_old)
        def _():
            acc_ref[...] = acc_ref[...] * jnp.exp(g_old - bmax)
            g_ref[0] = bmax

        g = g_ref[0]
        m_ref[...] = jnp.full((8, 128), g, jnp.float32)
        erow_ref[...] = jnp.exp(s - g).reshape(1, _B3)

    e_row = erow_ref[...]
    ids = ids_ref[0, 0, :]
    basew = base_ref[t]
    rel = ids - basew
    ohs = jnp.where(
        lax.broadcasted_iota(jnp.int32, (_SEGW, _B3), 0) == rel[None, :],
        e_row, 0.0).astype(jnp.bfloat16)
    part = lax.dot_general(
        ohs, f.astype(jnp.bfloat16), (((1,), (0,)), ((), ())),
        preferred_element_type=jnp.float32)
    acc_ref[pl.ds(basew, _SEGW), :] += part


def _fused_call(blks, bases, firsts, feats, w_col, b2, ids3):
    return pl.pallas_call(
        _fused_body,
        grid_spec=pltpu.PrefetchScalarGridSpec(
            num_scalar_prefetch=3,
            grid=(_TW,),
            in_specs=[
                pl.BlockSpec((_B3, _D), lambda t, blk, *_: (blk[t], 0)),
                pl.BlockSpec((_D, 1), lambda t, *_: (0, 0)),
                pl.BlockSpec((1, 1), lambda t, *_: (0, 0)),
                pl.BlockSpec((1, 1, _B3), lambda t, blk, *_: (blk[t], 0, 0)),
            ],
            out_specs=[
                pl.BlockSpec((_B3, 1), lambda t, blk, *_: (blk[t], 0)),
                pl.BlockSpec((_SEGP, _D), lambda t, *_: (0, 0)),
                pl.BlockSpec((8, 128), lambda t, *_: (0, 0)),
            ],
            scratch_shapes=[pltpu.SMEM((1,), jnp.float32),
                            pltpu.VMEM((1, _B3), jnp.float32)],
        ),
        out_shape=[
            jax.ShapeDtypeStruct((_N, 1), jnp.float32),
            jax.ShapeDtypeStruct((_SEGP, _D), jnp.float32),
            jax.ShapeDtypeStruct((8, 128), jnp.float32),
        ],
        compiler_params=pltpu.CompilerParams(
            dimension_semantics=("arbitrary",)),
    )(blks, bases, firsts, feats, w_col, b2, ids3)


def _sc_denom_partials_body(scores_hbm, ids_hbm, m_hbm, part_hbm,
                            sc_v, id_v, acc_v, m_v):
    cid = lax.axis_index("c")
    sid = lax.axis_index("s")
    wid = cid * _NS + sid
    base = wid * _RPW
    pltpu.sync_copy(scores_hbm.at[pl.ds(base, _RPW)], sc_v)
    pltpu.sync_copy(ids_hbm.at[pl.ds(base, _RPW)], id_v)
    pltpu.sync_copy(m_hbm.at[pl.ds(0, _L)], m_v)
    mvec = m_v[...]
    lane = lax.iota(jnp.int32, _L)
    shift = jnp.minimum(lane + 1, _L - 1)

    def zbody(i, _):
        acc_v[pl.ds(i * _L, _L)] = jnp.zeros((_L,), jnp.float32)
        return 0

    lax.fori_loop(0, _SEGP // _L, zbody, 0)

    def body(i, _):
        s = sc_v[pl.ds(i * _L, _L)]
        idx = id_v[pl.ds(i * _L, _L)]
        e = jnp.exp(s - mvec)
        cs = plsc.cumsum(e)
        idx_next = idx.at[shift].get(mode="promise_in_bounds")
        bnd = idx != idx_next
        is_last = bnd | (lane == _L - 1)
        plsc.addupdate_scatter(acc_v, [idx], cs, mask=is_last)
        plsc.addupdate_scatter(acc_v, [idx_next], -cs, mask=bnd)
        return 0

    lax.fori_loop(0, _RPW // _L, body, 0)
    pltpu.sync_copy(acc_v, part_hbm.at[wid])


@functools.lru_cache(maxsize=1)
def _sc_kernels():
    mesh = plsc.VectorSubcoreMesh(
        core_axis_name="c", subcore_axis_name="s",
        num_cores=_NC, num_subcores=_NS)
    denom_partials = pl.kernel(
        _sc_denom_partials_body,
        out_type=jax.ShapeDtypeStruct((_NW, _SEGP), jnp.float32),
        mesh=mesh,
        compiler_params=pltpu.CompilerParams(needs_layout_passes=False),
        scratch_types=[
            pltpu.VMEM((_RPW,), jnp.float32),
            pltpu.VMEM((_RPW,), jnp.int32),
            pltpu.VMEM((_SEGP,), jnp.float32),
            pltpu.VMEM((_L,), jnp.float32),
        ],
    )
    return denom_partials


def _reduce_body(part_ref, den_ref):
    s = jnp.sum(part_ref[...], axis=0, keepdims=True)
    den_ref[...] = jnp.broadcast_to(s, (8, den_ref.shape[1]))


def _reduce_call(part):
    return pl.pallas_call(
        _reduce_body,
        grid=(8,),
        in_specs=[pl.BlockSpec((_NW, _SEGP // 8), lambda k: (0, k))],
        out_specs=pl.BlockSpec((8, _SEGP // 8), lambda k: (0, k)),
        out_shape=jax.ShapeDtypeStruct((8, _SEGP), jnp.float32),
    )(part)


def _expand_body(acc_ref, dcol_ref, out_ref):
    j = pl.program_id(0)
    d = dcol_ref[...]
    inv = jnp.where(d > 0.0, 1.0 / d, 0.0)
    rows = acc_ref[...] * inv
    out_ref[...] = jnp.where(j < _SEGP // _B3, rows, jnp.zeros_like(rows))


def _expand_call(acc, dcol):
    nseg_blocks = _SEGP // _B3
    return pl.pallas_call(
        _expand_body,
        grid=(_NB3,),
        in_specs=[
            pl.BlockSpec((_B3, _D),
                         lambda j: (jnp.minimum(j, nseg_blocks - 1), 0)),
            pl.BlockSpec((_B3, 1),
                         lambda j: (jnp.minimum(j, nseg_blocks - 1), 0)),
        ],
        out_specs=pl.BlockSpec((_B3, _D), lambda j: (j, 0)),
        out_shape=jax.ShapeDtypeStruct((_N, _D), jnp.float32),
    )(acc, dcol)


def kernel(node_feats, batch_index, W, b):
    feats = node_feats.astype(jnp.float32)
    ids = batch_index.astype(jnp.int32)
    w_col = W.reshape(_D, 1).astype(jnp.float32)
    b2 = b.reshape(1, 1).astype(jnp.float32)

    ids_blk = ids.reshape(_NB3, _B3)
    w0s = (ids_blk[:, 0] // _SEGW).astype(jnp.int32)
    nwins = (ids_blk[:, -1] // _SEGW).astype(jnp.int32) - w0s + 1
    cum_incl = jnp.cumsum(nwins)
    cum_excl = cum_incl - nwins
    total = cum_incl[-1]
    t = jnp.arange(_TW, dtype=jnp.int32)
    blks = jnp.clip(jnp.searchsorted(cum_incl, t, side="right"),
                    0, _NB3 - 1).astype(jnp.int32)
    off = t - cum_excl[blks]
    valid = t < total
    bases = jnp.where(valid, (w0s[blks] + off) * _SEGW,
                      _SEGP - _SEGW).astype(jnp.int32)
    firsts = (valid & (off == 0)).astype(jnp.int32)

    scores2, acc, m = _fused_call(blks, bases, firsts, feats, w_col, b2,
                                  ids.reshape(_NB3, 1, _B3))
    scores = scores2.reshape(_N)
    m_flat = m.reshape(-1)

    part = _sc_kernels()(scores, ids, m_flat)
    den8 = _reduce_call(part)
    dcol = den8[0].reshape(_SEGP, 1)

    return _expand_call(acc, dcol)


# speedup vs baseline: 1.8987x; 1.8987x over previous
import functools

import jax
import jax.numpy as jnp
from jax import lax
from jax.experimental import pallas as pl
from jax.experimental.pallas import tpu as pltpu
from jax.experimental.pallas import tpu_sc as plsc

_N = 320000
_D = 128
_S = 10000

_B3 = 1280
_NB3 = _N // _B3
_SEGW = 64
_SEGP = 10240

_NC = 2
_NS = 16
_NW = _NC * _NS
_RPW = _N // _NW
_L = 16


def _fused_body(w0_ref, nwin_ref, feats_ref, w_ref, b_ref, ids_ref,
                scores_ref, acc_ref, m_ref, g_ref):
    k = pl.program_id(0)

    @pl.when(k == 0)
    def _():
        acc_ref[...] = jnp.zeros_like(acc_ref)
        g_ref[0] = -jnp.inf

    f = feats_ref[...]
    s_row = lax.dot_general(
        w_ref[...], f, (((1,), (1,)), ((), ())),
        preferred_element_type=jnp.float32) + b_ref[0, 0]
    scores_ref[0, 0, :] = s_row[0]
    bmax = jnp.max(s_row)
    g_old = g_ref[0]

    @pl.when(bmax > g_old)
    def _():
        acc_ref[...] = acc_ref[...] * jnp.exp(g_old - bmax)
        g_ref[0] = bmax

    g = g_ref[0]
    m_ref[...] = jnp.full((8, 128), g, jnp.float32)
    e_row = jnp.exp(s_row - g)
    ids = ids_ref[0, 0, :]
    f_bf = f.astype(jnp.bfloat16)
    w0 = w0_ref[k]
    nwin = nwin_ref[k]

    def wbody(o, _):
        basew = (w0 + o) * _SEGW
        rel = ids - basew
        ohs = jnp.where(
            lax.broadcasted_iota(jnp.int32, (_SEGW, _B3), 0) == rel[None, :],
            e_row, 0.0).astype(jnp.bfloat16)
        part = lax.dot_general(
            ohs, f_bf, (((1,), (0,)), ((), ())),
            preferred_element_type=jnp.float32)
        acc_ref[pl.ds(basew, _SEGW), :] += part
        return 0

    lax.fori_loop(0, nwin, wbody, 0)


def _fused_call(w0s, nwins, feats, w_row, b2, ids3):
    return pl.pallas_call(
        _fused_body,
        grid_spec=pltpu.PrefetchScalarGridSpec(
            num_scalar_prefetch=2,
            grid=(_NB3,),
            in_specs=[
                pl.BlockSpec((_B3, _D), lambda k, *_: (k, 0)),
                pl.BlockSpec((1, _D), lambda k, *_: (0, 0)),
                pl.BlockSpec((1, 1), lambda k, *_: (0, 0)),
                pl.BlockSpec((1, 1, _B3), lambda k, *_: (k, 0, 0)),
            ],
            out_specs=[
                pl.BlockSpec((1, 1, _B3), lambda k, *_: (k, 0, 0)),
                pl.BlockSpec((_SEGP, _D), lambda k, *_: (0, 0)),
                pl.BlockSpec((8, 128), lambda k, *_: (0, 0)),
            ],
            scratch_shapes=[pltpu.SMEM((1,), jnp.float32)],
        ),
        out_shape=[
            jax.ShapeDtypeStruct((_NB3, 1, _B3), jnp.float32),
            jax.ShapeDtypeStruct((_SEGP, _D), jnp.float32),
            jax.ShapeDtypeStruct((8, 128), jnp.float32),
        ],
        compiler_params=pltpu.CompilerParams(
            dimension_semantics=("arbitrary",)),
    )(w0s, nwins, feats, w_row, b2, ids3)


def _sc_denom_partials_body(scores_hbm, ids_hbm, m_hbm, part_hbm,
                            sc_v, id_v, acc_v, m_v):
    cid = lax.axis_index("c")
    sid = lax.axis_index("s")
    wid = cid * _NS + sid
    base = wid * _RPW
    pltpu.sync_copy(scores_hbm.at[pl.ds(base, _RPW)], sc_v)
    pltpu.sync_copy(ids_hbm.at[pl.ds(base, _RPW)], id_v)
    pltpu.sync_copy(m_hbm.at[pl.ds(0, _L)], m_v)
    mvec = m_v[...]
    lane = lax.iota(jnp.int32, _L)
    shift = jnp.minimum(lane + 1, _L - 1)

    def zbody(i, _):
        acc_v[pl.ds(i * _L, _L)] = jnp.zeros((_L,), jnp.float32)
        return 0

    lax.fori_loop(0, _SEGP // _L, zbody, 0)

    def body(i, _):
        s = sc_v[pl.ds(i * _L, _L)]
        idx = id_v[pl.ds(i * _L, _L)]
        e = jnp.exp(s - mvec)
        cs = plsc.cumsum(e)
        idx_next = idx.at[shift].get(mode="promise_in_bounds")
        bnd = idx != idx_next
        is_last = bnd | (lane == _L - 1)
        plsc.addupdate_scatter(acc_v, [idx], cs, mask=is_last)
        plsc.addupdate_scatter(acc_v, [idx_next], -cs, mask=bnd)
        return 0

    lax.fori_loop(0, _RPW // _L, body, 0)
    pltpu.sync_copy(acc_v, part_hbm.at[wid])


@functools.lru_cache(maxsize=1)
def _sc_kernels():
    mesh = plsc.VectorSubcoreMesh(
        core_axis_name="c", subcore_axis_name="s",
        num_cores=_NC, num_subcores=_NS)
    denom_partials = pl.kernel(
        _sc_denom_partials_body,
        out_type=jax.ShapeDtypeStruct((_NW, _SEGP), jnp.float32),
        mesh=mesh,
        compiler_params=pltpu.CompilerParams(needs_layout_passes=False),
        scratch_types=[
            pltpu.VMEM((_RPW,), jnp.float32),
            pltpu.VMEM((_RPW,), jnp.int32),
            pltpu.VMEM((_SEGP,), jnp.float32),
            pltpu.VMEM((_L,), jnp.float32),
        ],
    )
    return denom_partials


def _reduce_body(part_ref, den_ref):
    s = jnp.sum(part_ref[...], axis=0, keepdims=True)
    den_ref[...] = jnp.broadcast_to(s, (8, den_ref.shape[1]))


def _reduce_call(part):
    return pl.pallas_call(
        _reduce_body,
        grid=(8,),
        in_specs=[pl.BlockSpec((_NW, _SEGP // 8), lambda k: (0, k))],
        out_specs=pl.BlockSpec((8, _SEGP // 8), lambda k: (0, k)),
        out_shape=jax.ShapeDtypeStruct((8, _SEGP), jnp.float32),
    )(part)


def _expand_body(acc_ref, dcol_ref, out_ref):
    j = pl.program_id(0)
    d = dcol_ref[...]
    inv = jnp.where(d > 0.0, 1.0 / d, 0.0)
    rows = acc_ref[...] * inv
    out_ref[...] = jnp.where(j < _SEGP // _B3, rows, jnp.zeros_like(rows))


def _expand_call(acc, dcol):
    nseg_blocks = _SEGP // _B3
    return pl.pallas_call(
        _expand_body,
        grid=(_NB3,),
        in_specs=[
            pl.BlockSpec((_B3, _D),
                         lambda j: (jnp.minimum(j, nseg_blocks - 1), 0)),
            pl.BlockSpec((_B3, 1),
                         lambda j: (jnp.minimum(j, nseg_blocks - 1), 0)),
        ],
        out_specs=pl.BlockSpec((_B3, _D), lambda j: (j, 0)),
        out_shape=jax.ShapeDtypeStruct((_N, _D), jnp.float32),
    )(acc, dcol)


def kernel(node_feats, batch_index, W, b):
    feats = node_feats.astype(jnp.float32)
    ids = batch_index.astype(jnp.int32)
    w_col = W.reshape(_D, 1).astype(jnp.float32)
    b2 = b.reshape(1, 1).astype(jnp.float32)

    ids_blk = ids.reshape(_NB3, _B3)
    w0s = (ids_blk[:, 0] // _SEGW).astype(jnp.int32)
    nwins = (ids_blk[:, -1] // _SEGW).astype(jnp.int32) - w0s + 1

    scores3, acc, m = _fused_call(w0s, nwins, feats, w_col.reshape(1, _D),
                                  b2, ids.reshape(_NB3, 1, _B3))
    scores = scores3.reshape(_N)
    m_flat = m.reshape(-1)

    part = _sc_kernels()(scores, ids, m_flat)
    den8 = _reduce_call(part)
    dcol = den8[0].reshape(_SEGP, 1)

    return _expand_call(acc, dcol)


# speedup vs baseline: 2.6440x; 1.3926x over previous
import functools

import jax
import jax.numpy as jnp
from jax import lax
from jax.experimental import pallas as pl
from jax.experimental.pallas import tpu as pltpu
from jax.experimental.pallas import tpu_sc as plsc

_N = 320000
_D = 128
_S = 10000

_B3 = 2560
_NB3 = _N // _B3
_SEGW = 64
_SEGP = 10240

_NC = 2
_NS = 16
_NW = _NC * _NS
_RPW = _N // _NW
_L = 16


def _fused_body(w0_ref, nwin_ref, feats_ref, w_ref, b_ref, ids_ref,
                scores_ref, acc_ref, m_ref, g_ref):
    k = pl.program_id(0)

    @pl.when(k == 0)
    def _():
        acc_ref[...] = jnp.zeros_like(acc_ref)
        g_ref[0] = -jnp.inf

    f = feats_ref[...]
    s_row = lax.dot_general(
        w_ref[...], f, (((1,), (1,)), ((), ())),
        preferred_element_type=jnp.float32) + b_ref[0, 0]
    scores_ref[0, 0, :] = s_row[0]
    bmax = jnp.max(s_row)
    g_old = g_ref[0]

    @pl.when(bmax > g_old)
    def _():
        acc_ref[...] = acc_ref[...] * jnp.exp(g_old - bmax)
        g_ref[0] = bmax
        m_ref[...] = jnp.full((8, 128), bmax, jnp.float32)

    g = g_ref[0]
    e_row = jnp.exp(s_row - g)
    ids = ids_ref[0, 0, :]
    f_bf = f.astype(jnp.bfloat16)
    w0 = w0_ref[k]
    nwin = nwin_ref[k]

    def wbody(o, _):
        basew = (w0 + o) * _SEGW
        rel = ids - basew
        ohs = jnp.where(
            lax.broadcasted_iota(jnp.int32, (_SEGW, _B3), 0) == rel[None, :],
            e_row, 0.0).astype(jnp.bfloat16)
        part = lax.dot_general(
            ohs, f_bf, (((1,), (0,)), ((), ())),
            preferred_element_type=jnp.float32)
        acc_ref[pl.ds(basew, _SEGW), :] += part
        return 0

    lax.fori_loop(0, nwin, wbody, 0)


def _fused_call(w0s, nwins, feats, w_row, b2, ids3):
    return pl.pallas_call(
        _fused_body,
        grid_spec=pltpu.PrefetchScalarGridSpec(
            num_scalar_prefetch=2,
            grid=(_NB3,),
            in_specs=[
                pl.BlockSpec((_B3, _D), lambda k, *_: (k, 0)),
                pl.BlockSpec((1, _D), lambda k, *_: (0, 0)),
                pl.BlockSpec((1, 1), lambda k, *_: (0, 0)),
                pl.BlockSpec((1, 1, _B3), lambda k, *_: (k, 0, 0)),
            ],
            out_specs=[
                pl.BlockSpec((1, 1, _B3), lambda k, *_: (k, 0, 0)),
                pl.BlockSpec((_SEGP, _D), lambda k, *_: (0, 0)),
                pl.BlockSpec((8, 128), lambda k, *_: (0, 0)),
            ],
            scratch_shapes=[pltpu.SMEM((1,), jnp.float32)],
        ),
        out_shape=[
            jax.ShapeDtypeStruct((_NB3, 1, _B3), jnp.float32),
            jax.ShapeDtypeStruct((_SEGP, _D), jnp.float32),
            jax.ShapeDtypeStruct((8, 128), jnp.float32),
        ],
        compiler_params=pltpu.CompilerParams(
            dimension_semantics=("arbitrary",)),
    )(w0s, nwins, feats, w_row, b2, ids3)


def _sc_denom_partials_body(scores_hbm, ids_hbm, m_hbm, part_hbm,
                            sc_v, id_v, acc_v, m_v):
    cid = lax.axis_index("c")
    sid = lax.axis_index("s")
    wid = cid * _NS + sid
    base = wid * _RPW
    pltpu.sync_copy(scores_hbm.at[pl.ds(base, _RPW)], sc_v)
    pltpu.sync_copy(ids_hbm.at[pl.ds(base, _RPW)], id_v)
    pltpu.sync_copy(m_hbm.at[pl.ds(0, _L)], m_v)
    mvec = m_v[...]
    lane = lax.iota(jnp.int32, _L)
    shift = jnp.minimum(lane + 1, _L - 1)

    def zbody(i, _):
        acc_v[pl.ds(i * _L, _L)] = jnp.zeros((_L,), jnp.float32)
        return 0

    lax.fori_loop(0, _SEGP // _L, zbody, 0)

    def body(i, _):
        s = sc_v[pl.ds(i * _L, _L)]
        idx = id_v[pl.ds(i * _L, _L)]
        e = jnp.exp(s - mvec)
        cs = plsc.cumsum(e)
        idx_next = idx.at[shift].get(mode="promise_in_bounds")
        bnd = idx != idx_next
        is_last = bnd | (lane == _L - 1)
        plsc.addupdate_scatter(acc_v, [idx], cs, mask=is_last)
        plsc.addupdate_scatter(acc_v, [idx_next], -cs, mask=bnd)
        return 0

    lax.fori_loop(0, _RPW // _L, body, 0)
    pltpu.sync_copy(acc_v, part_hbm.at[wid])


@functools.lru_cache(maxsize=1)
def _sc_kernels():
    mesh = plsc.VectorSubcoreMesh(
        core_axis_name="c", subcore_axis_name="s",
        num_cores=_NC, num_subcores=_NS)
    denom_partials = pl.kernel(
        _sc_denom_partials_body,
        out_type=jax.ShapeDtypeStruct((_NW, _SEGP), jnp.float32),
        mesh=mesh,
        compiler_params=pltpu.CompilerParams(needs_layout_passes=False),
        scratch_types=[
            pltpu.VMEM((_RPW,), jnp.float32),
            pltpu.VMEM((_RPW,), jnp.int32),
            pltpu.VMEM((_SEGP,), jnp.float32),
            pltpu.VMEM((_L,), jnp.float32),
        ],
    )
    return denom_partials


def _reduce_body(part_ref, den_ref):
    s = jnp.sum(part_ref[...], axis=0, keepdims=True)
    den_ref[...] = jnp.broadcast_to(s, (8, den_ref.shape[1]))


def _reduce_call(part):
    return pl.pallas_call(
        _reduce_body,
        grid=(8,),
        in_specs=[pl.BlockSpec((_NW, _SEGP // 8), lambda k: (0, k))],
        out_specs=pl.BlockSpec((8, _SEGP // 8), lambda k: (0, k)),
        out_shape=jax.ShapeDtypeStruct((8, _SEGP), jnp.float32),
    )(part)


def _expand_body(acc_ref, dcol_ref, out_ref):
    j = pl.program_id(0)
    d = dcol_ref[...]
    inv = jnp.where(d > 0.0, 1.0 / d, 0.0)
    rows = acc_ref[...] * inv
    out_ref[...] = jnp.where(j < _SEGP // _B3, rows, jnp.zeros_like(rows))


def _expand_call(acc, dcol):
    nseg_blocks = _SEGP // _B3
    return pl.pallas_call(
        _expand_body,
        grid=(_NB3,),
        in_specs=[
            pl.BlockSpec((_B3, _D),
                         lambda j: (jnp.minimum(j, nseg_blocks - 1), 0)),
            pl.BlockSpec((_B3, 1),
                         lambda j: (jnp.minimum(j, nseg_blocks - 1), 0)),
        ],
        out_specs=pl.BlockSpec((_B3, _D), lambda j: (j, 0)),
        out_shape=jax.ShapeDtypeStruct((_N, _D), jnp.float32),
    )(acc, dcol)


def kernel(node_feats, batch_index, W, b):
    feats = node_feats.astype(jnp.float32)
    ids = batch_index.astype(jnp.int32)
    w_col = W.reshape(_D, 1).astype(jnp.float32)
    b2 = b.reshape(1, 1).astype(jnp.float32)

    ids_blk = ids.reshape(_NB3, _B3)
    w0s = (ids_blk[:, 0] // _SEGW).astype(jnp.int32)
    nwins = (ids_blk[:, -1] // _SEGW).astype(jnp.int32) - w0s + 1

    scores3, acc, m = _fused_call(w0s, nwins, feats, w_col.reshape(1, _D),
                                  b2, ids.reshape(_NB3, 1, _B3))
    scores = scores3.reshape(_N)
    m_flat = m.reshape(-1)

    part = _sc_kernels()(scores, ids, m_flat)
    den8 = _reduce_call(part)
    dcol = den8[0].reshape(_SEGP, 1)

    return _expand_call(acc, dcol)
